# Initial kernel scaffold; baseline (speedup 1.0000x reference)
import jax
import jax.numpy as jnp
from jax.experimental import pallas as pl


def kernel(x, edge_index, edge_weight, W_xz, b_xz, W_hz, b_hz, W_xr, b_xr, W_hr, b_hr, W_xh, b_xh, W_hh, b_hh, W_lin, b_lin):
    raise NotImplementedError("write your pallas kernel here")


# speedup vs baseline: 1.0516x; 1.0516x over previous
import jax
import jax.numpy as jnp
from jax.experimental import pallas as pl
from jax.experimental.pallas import tpu as pltpu

_D = 128
_BM = 1000


def _fused(x_ref, w_ref, cz_ref, ch_ref, wlin_ref, blin_ref, o_ref):
    y = jnp.dot(x_ref[...], w_ref[...], preferred_element_type=jnp.float32)
    z = jax.nn.sigmoid(y[:, :_D] + cz_ref[...])
    ht = jnp.tanh(y[:, _D:] + ch_ref[...])
    g = jax.nn.relu((1.0 - z) * ht)
    o_ref[...] = (
        jnp.dot(g, wlin_ref[...], preferred_element_type=jnp.float32)
        + blin_ref[...]
    )


def kernel(x, edge_index, edge_weight, W_xz, b_xz, W_hz, b_hz, W_xr, b_xr,
           W_hr, b_hr, W_xh, b_xh, W_hh, b_hh, W_lin, b_lin):
    n = x.shape[0]
    w_cat = jnp.concatenate([W_xz, W_xh], axis=1)
    cz = (b_xz + b_hz).reshape(1, _D)
    ch = (b_xh + b_hh).reshape(1, _D)
    blin = b_lin.reshape(1, 1)

    grid = (n // _BM,)
    return pl.pallas_call(
        _fused,
        grid=grid,
        in_specs=[
            pl.BlockSpec((_BM, _D), lambda i: (i, 0)),
            pl.BlockSpec((_D, 2 * _D), lambda i: (0, 0)),
            pl.BlockSpec((1, _D), lambda i: (0, 0)),
            pl.BlockSpec((1, _D), lambda i: (0, 0)),
            pl.BlockSpec((_D, 1), lambda i: (0, 0)),
            pl.BlockSpec((1, 1), lambda i: (0, 0)),
        ],
        out_specs=pl.BlockSpec((_BM, 1), lambda i: (i, 0)),
        out_shape=jax.ShapeDtypeStruct((n, 1), x.dtype),
        compiler_params=pltpu.CompilerParams(
            dimension_semantics=("arbitrary",),
        ),
    )(x, w_cat, cz, ch, W_lin, blin)


# speedup vs baseline: 1.1821x; 1.1240x over previous
import jax
import jax.numpy as jnp
from jax.experimental import pallas as pl
from jax.experimental.pallas import tpu as pltpu

_D = 128
_BM = 1000


def _fused(x_ref, wz_ref, wh_ref, bxz_ref, bhz_ref, bxh_ref, bhh_ref,
           wlin_ref, blin_ref, o_ref):
    x = x_ref[...]
    z = jax.nn.sigmoid(
        jnp.dot(x, wz_ref[...], preferred_element_type=jnp.float32)
        + (bxz_ref[...] + bhz_ref[...])
    )
    ht = jnp.tanh(
        jnp.dot(x, wh_ref[...], preferred_element_type=jnp.float32)
        + (bxh_ref[...] + bhh_ref[...])
    )
    g = jax.nn.relu((1.0 - z) * ht)
    o_ref[...] = (
        jnp.dot(g, wlin_ref[...], preferred_element_type=jnp.float32)
        + blin_ref[...]
    )


def kernel(x, edge_index, edge_weight, W_xz, b_xz, W_hz, b_hz, W_xr, b_xr,
           W_hr, b_hr, W_xh, b_xh, W_hh, b_hh, W_lin, b_lin):
    n = x.shape[0]
    bxz = b_xz.reshape(1, _D)
    bhz = b_hz.reshape(1, _D)
    bxh = b_xh.reshape(1, _D)
    bhh = b_hh.reshape(1, _D)
    blin = b_lin.reshape(1, 1)

    grid = (n // _BM,)
    return pl.pallas_call(
        _fused,
        grid=grid,
        in_specs=[
            pl.BlockSpec((_BM, _D), lambda i: (i, 0)),
            pl.BlockSpec((_D, _D), lambda i: (0, 0)),
            pl.BlockSpec((_D, _D), lambda i: (0, 0)),
            pl.BlockSpec((1, _D), lambda i: (0, 0)),
            pl.BlockSpec((1, _D), lambda i: (0, 0)),
            pl.BlockSpec((1, _D), lambda i: (0, 0)),
            pl.BlockSpec((1, _D), lambda i: (0, 0)),
            pl.BlockSpec((_D, 1), lambda i: (0, 0)),
            pl.BlockSpec((1, 1), lambda i: (0, 0)),
        ],
        out_specs=pl.BlockSpec((_BM, 1), lambda i: (i, 0)),
        out_shape=jax.ShapeDtypeStruct((n, 1), x.dtype),
        compiler_params=pltpu.CompilerParams(
            dimension_semantics=("arbitrary",),
        ),
    )(x, W_xz, W_xh, bxz, bhz, bxh, bhh, W_lin, blin)


# speedup vs baseline: 1.4481x; 1.2251x over previous
import jax
import jax.numpy as jnp
from jax.experimental import pallas as pl
from jax.experimental.pallas import tpu as pltpu

_D = 128
_BM = 2000


def _fused(x_ref, wz_ref, wh_ref, bxz_ref, bhz_ref, bxh_ref, bhh_ref,
           wlin_ref, blin_ref, o_ref):
    x = x_ref[...]
    z = jax.nn.sigmoid(
        jnp.dot(x, wz_ref[...], preferred_element_type=jnp.float32)
        + (bxz_ref[...] + bhz_ref[...])
    )
    ht = jnp.tanh(
        jnp.dot(x, wh_ref[...], preferred_element_type=jnp.float32)
        + (bxh_ref[...] + bhh_ref[...])
    )
    g = jax.nn.relu((1.0 - z) * ht)
    o_ref[...] = (
        jnp.dot(g, wlin_ref[...], preferred_element_type=jnp.float32)
        + blin_ref[...]
    )


def kernel(x, edge_index, edge_weight, W_xz, b_xz, W_hz, b_hz, W_xr, b_xr,
           W_hr, b_hr, W_xh, b_xh, W_hh, b_hh, W_lin, b_lin):
    n = x.shape[0]
    bxz = b_xz.reshape(1, _D)
    bhz = b_hz.reshape(1, _D)
    bxh = b_xh.reshape(1, _D)
    bhh = b_hh.reshape(1, _D)
    blin = b_lin.reshape(1, 1)

    grid = (n // _BM,)
    return pl.pallas_call(
        _fused,
        grid=grid,
        in_specs=[
            pl.BlockSpec((_BM, _D), lambda i: (i, 0)),
            pl.BlockSpec((_D, _D), lambda i: (0, 0)),
            pl.BlockSpec((_D, _D), lambda i: (0, 0)),
            pl.BlockSpec((1, _D), lambda i: (0, 0)),
            pl.BlockSpec((1, _D), lambda i: (0, 0)),
            pl.BlockSpec((1, _D), lambda i: (0, 0)),
            pl.BlockSpec((1, _D), lambda i: (0, 0)),
            pl.BlockSpec((_D, 1), lambda i: (0, 0)),
            pl.BlockSpec((1, 1), lambda i: (0, 0)),
        ],
        out_specs=pl.BlockSpec((_BM, 1), lambda i: (i, 0)),
        out_shape=jax.ShapeDtypeStruct((n, 1), x.dtype),
        compiler_params=pltpu.CompilerParams(
            dimension_semantics=("parallel",),
        ),
    )(x, W_xz, W_xh, bxz, bhz, bxh, bhh, W_lin, blin)


# speedup vs baseline: 1.4577x; 1.0066x over previous
import jax
import jax.numpy as jnp
from jax.experimental import pallas as pl
from jax.experimental.pallas import tpu as pltpu

_D = 128
_BM = 5000


def _fused(x_ref, wz_ref, wh_ref, bxz_ref, bhz_ref, bxh_ref, bhh_ref,
           wlin_ref, blin_ref, o_ref):
    x = x_ref[...]
    z = jax.nn.sigmoid(
        jnp.dot(x, wz_ref[...], preferred_element_type=jnp.float32)
        + (bxz_ref[...] + bhz_ref[...])
    )
    ht = jnp.tanh(
        jnp.dot(x, wh_ref[...], preferred_element_type=jnp.float32)
        + (bxh_ref[...] + bhh_ref[...])
    )
    g = jax.nn.relu((1.0 - z) * ht)
    o_ref[...] = (
        jnp.dot(g, wlin_ref[...], preferred_element_type=jnp.float32)
        + blin_ref[...]
    )


def kernel(x, edge_index, edge_weight, W_xz, b_xz, W_hz, b_hz, W_xr, b_xr,
           W_hr, b_hr, W_xh, b_xh, W_hh, b_hh, W_lin, b_lin):
    n = x.shape[0]
    bxz = b_xz.reshape(1, _D)
    bhz = b_hz.reshape(1, _D)
    bxh = b_xh.reshape(1, _D)
    bhh = b_hh.reshape(1, _D)
    blin = b_lin.reshape(1, 1)

    grid = (n // _BM,)
    return pl.pallas_call(
        _fused,
        grid=grid,
        in_specs=[
            pl.BlockSpec((_BM, _D), lambda i: (i, 0)),
            pl.BlockSpec((_D, _D), lambda i: (0, 0)),
            pl.BlockSpec((_D, _D), lambda i: (0, 0)),
            pl.BlockSpec((1, _D), lambda i: (0, 0)),
            pl.BlockSpec((1, _D), lambda i: (0, 0)),
            pl.BlockSpec((1, _D), lambda i: (0, 0)),
            pl.BlockSpec((1, _D), lambda i: (0, 0)),
            pl.BlockSpec((_D, 1), lambda i: (0, 0)),
            pl.BlockSpec((1, 1), lambda i: (0, 0)),
        ],
        out_specs=pl.BlockSpec((_BM, 1), lambda i: (i, 0)),
        out_shape=jax.ShapeDtypeStruct((n, 1), x.dtype),
        compiler_params=pltpu.CompilerParams(
            dimension_semantics=("parallel",),
        ),
    )(x, W_xz, W_xh, bxz, bhz, bxh, bhh, W_lin, blin)


# speedup vs baseline: 1.5089x; 1.0351x over previous
import jax
import jax.numpy as jnp
from jax.experimental import pallas as pl
from jax.experimental.pallas import tpu as pltpu

_D = 128
_BM = 10000


def _fused(x_ref, wz_ref, wh_ref, bxz_ref, bhz_ref, bxh_ref, bhh_ref,
           wlin_ref, blin_ref, o_ref):
    x = x_ref[...]
    z = jax.nn.sigmoid(
        jnp.dot(x, wz_ref[...], preferred_element_type=jnp.float32)
        + (bxz_ref[...] + bhz_ref[...])
    )
    ht = jnp.tanh(
        jnp.dot(x, wh_ref[...], preferred_element_type=jnp.float32)
        + (bxh_ref[...] + bhh_ref[...])
    )
    g = jax.nn.relu((1.0 - z) * ht)
    o_ref[...] = (
        jnp.dot(g, wlin_ref[...], preferred_element_type=jnp.float32)
        + blin_ref[...]
    )


def kernel(x, edge_index, edge_weight, W_xz, b_xz, W_hz, b_hz, W_xr, b_xr,
           W_hr, b_hr, W_xh, b_xh, W_hh, b_hh, W_lin, b_lin):
    n = x.shape[0]
    bxz = b_xz.reshape(1, _D)
    bhz = b_hz.reshape(1, _D)
    bxh = b_xh.reshape(1, _D)
    bhh = b_hh.reshape(1, _D)
    blin = b_lin.reshape(1, 1)

    grid = (n // _BM,)
    return pl.pallas_call(
        _fused,
        grid=grid,
        in_specs=[
            pl.BlockSpec((_BM, _D), lambda i: (i, 0)),
            pl.BlockSpec((_D, _D), lambda i: (0, 0)),
            pl.BlockSpec((_D, _D), lambda i: (0, 0)),
            pl.BlockSpec((1, _D), lambda i: (0, 0)),
            pl.BlockSpec((1, _D), lambda i: (0, 0)),
            pl.BlockSpec((1, _D), lambda i: (0, 0)),
            pl.BlockSpec((1, _D), lambda i: (0, 0)),
            pl.BlockSpec((_D, 1), lambda i: (0, 0)),
            pl.BlockSpec((1, 1), lambda i: (0, 0)),
        ],
        out_specs=pl.BlockSpec((_BM, 1), lambda i: (i, 0)),
        out_shape=jax.ShapeDtypeStruct((n, 1), x.dtype),
        compiler_params=pltpu.CompilerParams(
            dimension_semantics=("parallel",),
        ),
    )(x, W_xz, W_xh, bxz, bhz, bxh, bhh, W_lin, blin)
